# baseline (speedup 1.0000x reference)
import jax
import jax.numpy as jnp
from jax import lax
from jax.experimental import pallas as pl
from jax.experimental.pallas import tpu as pltpu
from jax.experimental.pallas import tpu_sc as plsc

B, T, C, P, R = 16384, 32, 8, 3, 9
NC, NS, L = 2, 16, 16
NW = NC * NS
ROWS = B // NW
GROUPS = ROWS // L
TR = T // 8
TB = B // 128
WTB = ROWS // 128
PAR_RE = 32


_LOG1P = (9.999987639e-01, -4.998719253e-01, 3.311205837e-01,
          -2.351488241e-01, 1.494348365e-01, -6.658820573e-02,
          1.420285926e-02)


def _softplus(x):
  t = jnp.exp(-jnp.abs(x))
  q = _LOG1P[6]
  for cc in _LOG1P[5::-1]:
    q = q * t + cc
  return jnp.maximum(x, 0.0) + t * q


def _body(ns_hbm, ch_hbm, rl_hbm, par_hbm, out_hbm,
          ns_v, ch_v, rl_v, par_v, chain_v, mlen_v, sem_ns, sem_ch, sem_rl):
  cid = lax.axis_index("c")
  sid = lax.axis_index("s")
  wid = sid * NC + cid
  tb0 = wid * WTB

  lane = lax.broadcasted_iota(jnp.int32, (L,), 0)

  cp_ns = pltpu.async_copy(ns_hbm.at[:, :, pl.ds(tb0, WTB)], ns_v, sem_ns)
  cp_ch = pltpu.async_copy(ch_hbm.at[:, pl.ds(tb0, WTB), C - 1], ch_v, sem_ch)
  cp_rl = pltpu.async_copy(rl_hbm.at[:, pl.ds(tb0, WTB), C - 1], rl_v, sem_rl)
  pltpu.sync_copy(par_hbm, par_v)
  cp_ch.wait()

  wv = par_v[pl.ds(0, L)]
  vv = par_v[pl.ds(L, L)]
  w = [[wv[q * P + p] for p in range(P)] for q in range(P)]
  v = [[vv[q * P + p] for p in range(P)] for q in range(P)]

  @plsc.parallel_loop(0, GROUPS)
  def _walk(g):
    tcl = g >> 3
    l0 = (g & 7) * L
    tcl_v = jnp.full((L,), 0, jnp.int32) + tcl
    lane_v = l0 + lane
    g_v = jnp.full((L,), 0, jnp.int32) + g

    def cond(carry):
      _, _, _, act = carry
      return jnp.any(act)

    def body(carry):
      pos, mlen, k, act = carry
      plsc.store_scatter(chain_v, [g_v, jnp.full((L,), 0, jnp.int32) + k,
                                   lane], pos, mask=act)
      c = plsc.load_gather(ch_v, [pos, tcl_v, lane_v])
      act2 = jnp.logical_and(act, c < pos)
      pos2 = jnp.where(act2, c, pos)
      return pos2, mlen + act2.astype(jnp.int32), k + 1, act2

    _, mlen, _, _ = lax.while_loop(
        cond, body,
        (jnp.full((L,), T - 1, jnp.int32), jnp.full((L,), 1, jnp.int32),
         jnp.int32(0), lane < L))
    mlen_v[g, :] = mlen

  def max_step(gg, acc):
    return jnp.maximum(acc, mlen_v[gg, :])

  mmax = jnp.max(lax.fori_loop(0, GROUPS, max_step,
                               jnp.full((L,), 1, jnp.int32)))
  cp_ns.wait()
  cp_rl.wait()

  def eval_kk(kk, carry):
    k = mmax - 1 - kk

    @plsc.parallel_loop(0, GROUPS, unroll=4)
    def _group(g):
      tcl = g >> 3
      l0 = (g & 7) * L
      tcl_v = jnp.full((L,), 0, jnp.int32) + tcl
      lane_v = l0 + lane
      valid = k < mlen_v[g, pl.ds(0, L)]
      pos = chain_v[g, k, pl.ds(0, L)]
      pos = jnp.minimum(jnp.maximum(pos, 0), T - 1)
      ptr = lax.shift_right_logical(pos, 3)
      pr = jnp.bitwise_and(pos, 7)
      c = plsc.load_gather(ch_v, [pos, tcl_v, lane_v])
      r = plsc.load_gather(rl_v, [pos, tcl_v, lane_v])
      ctr = lax.shift_right_logical(c, 3)
      cr = jnp.bitwise_and(c, 7)
      prnt = [plsc.load_gather(
          ns_v, [jnp.full((L,), p, jnp.int32), ptr, tcl_v, pr, lane_v])
              for p in range(P)]
      cs = [plsc.load_gather(
          ns_v, [jnp.full((L,), p, jnp.int32), ctr, tcl_v, cr, lane_v])
            for p in range(P)]
      re = [plsc.load_gather(par_v, [r * P + (PAR_RE + p)]) for p in range(P)]
      a = []
      for p in range(P):
        x = re[p]
        for q in range(P):
          x = x + w[q][p] * prnt[q]
          x = x + v[q][p] * cs[q]
        a.append(_softplus(x) + 1e-6)
      inv = 1.0 / (a[0] + a[1] + a[2])
      for p in range(P):
        plsc.store_scatter(
            ns_v, [jnp.full((L,), p, jnp.int32), ptr, tcl_v, pr, lane_v],
            a[p] * inv, mask=valid)

    return carry

  lax.fori_loop(0, mmax, eval_kk, 0)

  pltpu.sync_copy(ns_v.at[:, TR - 1, :, 7, :],
                  out_hbm.at[:, pl.ds(tb0, WTB)])


@jax.jit
def _run(ns5, ch4, rl4, params):
  mesh = plsc.VectorSubcoreMesh(core_axis_name="c", subcore_axis_name="s")
  f = pl.kernel(
      _body,
      out_type=jax.ShapeDtypeStruct((P, TB, 128), jnp.float32),
      mesh=mesh,
      scratch_types=[
          pltpu.VMEM((P, TR, WTB, 8, 128), jnp.float32),
          pltpu.VMEM((T, WTB, 128), jnp.int32),
          pltpu.VMEM((T, WTB, 128), jnp.int32),
          pltpu.VMEM((64,), jnp.float32),
          pltpu.VMEM((GROUPS, T, L), jnp.int32),
          pltpu.VMEM((GROUPS, L), jnp.int32),
          pltpu.SemaphoreType.DMA,
          pltpu.SemaphoreType.DMA,
          pltpu.SemaphoreType.DMA,
      ],
      compiler_params=pltpu.CompilerParams(
          needs_layout_passes=False, use_tc_tiling_on_sc=False,
          skip_device_barrier=True),
  )
  return f(ns5, ch4, rl4, params)


def kernel(node_scores, children, rels, labels, W, V, rel_emb):
  del labels
  ns5 = (node_scores.transpose(2, 1, 0)
         .reshape(P, TR, 8, TB, 128)
         .transpose(0, 1, 3, 2, 4))
  ch4 = (children.astype(jnp.int32).transpose(1, 2, 0)
         .reshape(T, C, TB, 128)
         .transpose(0, 2, 1, 3))
  rl4 = (rels.astype(jnp.int32).transpose(1, 2, 0)
         .reshape(T, C, TB, 128)
         .transpose(0, 2, 1, 3))
  params = (jnp.zeros((64,), jnp.float32)
            .at[0:P * P].set(W.reshape(-1))
            .at[L:L + P * P].set(V.reshape(-1))
            .at[PAR_RE:PAR_RE + R * P].set(rel_emb.reshape(-1)))
  out = _run(ns5, ch4, rl4, params)
  return out.reshape(P, B).transpose(1, 0)


# speedup vs baseline: 1.0026x; 1.0026x over previous
import jax
import jax.numpy as jnp
from jax import lax
from jax.experimental import pallas as pl
from jax.experimental.pallas import tpu as pltpu
from jax.experimental.pallas import tpu_sc as plsc

B, T, C, P, R = 16384, 32, 8, 3, 9
NC, NS, L = 2, 16, 16
NW = NC * NS
ROWS = B // NW
GROUPS = ROWS // L
TR = T // 8
TB = B // 128
WTB = ROWS // 128
PAR_RE = 32


_LOG1P = (9.999987639e-01, -4.998719253e-01, 3.311205837e-01,
          -2.351488241e-01, 1.494348365e-01, -6.658820573e-02,
          1.420285926e-02)


def _softplus(x):
  t = jnp.exp(-jnp.abs(x))
  q = _LOG1P[6]
  for cc in _LOG1P[5::-1]:
    q = q * t + cc
  return jnp.maximum(x, 0.0) + t * q


def _body(ns_hbm, ch_hbm, rl_hbm, par_hbm, out_hbm,
          ns_v, ch_v, rl_v, par_v, chain_v, mlen_v, sem_ns, sem_ch, sem_rl):
  cid = lax.axis_index("c")
  sid = lax.axis_index("s")
  wid = sid * NC + cid
  tb0 = wid * WTB

  lane = lax.broadcasted_iota(jnp.int32, (L,), 0)

  cp_ns = pltpu.async_copy(ns_hbm.at[:, :, pl.ds(tb0, WTB)], ns_v, sem_ns)
  cp_ch = pltpu.async_copy(ch_hbm.at[:, pl.ds(tb0, WTB), C - 1], ch_v, sem_ch)
  cp_rl = pltpu.async_copy(rl_hbm.at[:, pl.ds(tb0, WTB), C - 1], rl_v, sem_rl)
  pltpu.sync_copy(par_hbm, par_v)
  cp_ch.wait()

  wv = par_v[pl.ds(0, L)]
  vv = par_v[pl.ds(L, L)]
  w = [[wv[q * P + p] for p in range(P)] for q in range(P)]
  v = [[vv[q * P + p] for p in range(P)] for q in range(P)]

  @plsc.parallel_loop(0, GROUPS)
  def _walk(g):
    tcl = g >> 3
    l0 = (g & 7) * L
    tcl_v = jnp.full((L,), 0, jnp.int32) + tcl
    lane_v = l0 + lane
    g_v = jnp.full((L,), 0, jnp.int32) + g

    def cond(carry):
      _, _, _, act = carry
      return jnp.any(act)

    def body(carry):
      pos, mlen, k, act = carry
      plsc.store_scatter(chain_v, [g_v, jnp.full((L,), 0, jnp.int32) + k,
                                   lane], pos, mask=act)
      c = plsc.load_gather(ch_v, [pos, tcl_v, lane_v])
      act2 = jnp.logical_and(act, c < pos)
      pos2 = jnp.where(act2, c, pos)
      return pos2, mlen + act2.astype(jnp.int32), k + 1, act2

    _, mlen, _, _ = lax.while_loop(
        cond, body,
        (jnp.full((L,), T - 1, jnp.int32), jnp.full((L,), 1, jnp.int32),
         jnp.int32(0), lane < L))
    mlen_v[g, :] = mlen

  def max_step(gg, acc):
    return jnp.maximum(acc, mlen_v[gg, :])

  mmax = jnp.max(lax.fori_loop(0, GROUPS, max_step,
                               jnp.full((L,), 1, jnp.int32)))
  cp_ns.wait()
  cp_rl.wait()

  def eval_kk(kk, carry):
    k = mmax - 1 - kk

    @plsc.parallel_loop(0, GROUPS, unroll=4)
    def _group(g):
      tcl = g >> 3
      l0 = (g & 7) * L
      tcl_v = jnp.full((L,), 0, jnp.int32) + tcl
      lane_v = l0 + lane
      valid = k < mlen_v[g, pl.ds(0, L)]
      pos = chain_v[g, k, pl.ds(0, L)]
      pos = jnp.minimum(jnp.maximum(pos, 0), T - 1)
      ptr = lax.shift_right_logical(pos, 3)
      pr = jnp.bitwise_and(pos, 7)
      c = plsc.load_gather(ch_v, [pos, tcl_v, lane_v])
      r = plsc.load_gather(rl_v, [pos, tcl_v, lane_v])
      ctr = lax.shift_right_logical(c, 3)
      cr = jnp.bitwise_and(c, 7)
      prnt = [plsc.load_gather(
          ns_v, [jnp.full((L,), p, jnp.int32), ptr, tcl_v, pr, lane_v])
              for p in range(P)]
      cs = [plsc.load_gather(
          ns_v, [jnp.full((L,), p, jnp.int32), ctr, tcl_v, cr, lane_v])
            for p in range(P)]
      re = [plsc.load_gather(par_v, [r * P + (PAR_RE + p)]) for p in range(P)]
      a = []
      for p in range(P):
        x = re[p]
        for q in range(P):
          x = x + w[q][p] * prnt[q]
          x = x + v[q][p] * cs[q]
        a.append(_softplus(x) + 1e-6)
      inv = 1.0 / (a[0] + a[1] + a[2])
      for p in range(P):
        plsc.store_scatter(
            ns_v, [jnp.full((L,), p, jnp.int32), ptr, tcl_v, pr, lane_v],
            a[p] * inv, mask=valid)

    return carry

  lax.fori_loop(0, mmax, eval_kk, 0)

  pltpu.sync_copy(ns_v.at[:, TR - 1, :, 7, :],
                  out_hbm.at[:, pl.ds(tb0, WTB)])


@jax.jit
def _run(ns5, ch4, rl4, params):
  mesh = plsc.VectorSubcoreMesh(core_axis_name="c", subcore_axis_name="s")
  f = pl.kernel(
      _body,
      out_type=jax.ShapeDtypeStruct((P, TB, 128), jnp.float32),
      mesh=mesh,
      scratch_types=[
          pltpu.VMEM((P, TR, WTB, 8, 128), jnp.float32),
          pltpu.VMEM((T, WTB, 128), jnp.int32),
          pltpu.VMEM((T, WTB, 128), jnp.int32),
          pltpu.VMEM((64,), jnp.float32),
          pltpu.VMEM((GROUPS, T, L), jnp.int32),
          pltpu.VMEM((GROUPS, L), jnp.int32),
          pltpu.SemaphoreType.DMA,
          pltpu.SemaphoreType.DMA,
          pltpu.SemaphoreType.DMA,
      ],
      compiler_params=pltpu.CompilerParams(
          needs_layout_passes=False, use_tc_tiling_on_sc=False),
  )
  return f(ns5, ch4, rl4, params)


def kernel(node_scores, children, rels, labels, W, V, rel_emb):
  del labels
  ns5 = (node_scores.transpose(2, 1, 0)
         .reshape(P, TR, 8, TB, 128)
         .transpose(0, 1, 3, 2, 4))
  ch4 = (children.astype(jnp.int32).transpose(1, 2, 0)
         .reshape(T, C, TB, 128)
         .transpose(0, 2, 1, 3))
  rl4 = (rels.astype(jnp.int32).transpose(1, 2, 0)
         .reshape(T, C, TB, 128)
         .transpose(0, 2, 1, 3))
  params = (jnp.zeros((64,), jnp.float32)
            .at[0:P * P].set(W.reshape(-1))
            .at[L:L + P * P].set(V.reshape(-1))
            .at[PAR_RE:PAR_RE + R * P].set(rel_emb.reshape(-1)))
  out = _run(ns5, ch4, rl4, params)
  return out.reshape(P, B).transpose(1, 0)
